# baseline (speedup 1.0000x reference)
import functools

import jax
import jax.numpy as jnp
from jax import lax
from jax.experimental import pallas as pl
from jax.experimental.pallas import tpu as pltpu
from jax.experimental.pallas import tpu_sc as plsc

_B = 16384
_C = 16
_S = 30.0
_LN2 = 0.6931471805599453
_CHUNK = 128


_LOG2E = 1.4426950408889634
_MAGIC = 12582912.0
_MAGIC_I = 0x4B400000


def _fast_exp(v):
    a = jnp.maximum(v * _LOG2E, -126.0)
    t = a + _MAGIC
    kf = t - _MAGIC
    f = a - kf
    ik = lax.bitcast_convert_type(t, jnp.int32) - _MAGIC_I
    p2k = lax.bitcast_convert_type((ik + 127) << 23, jnp.float32)
    w = f * _LN2
    poly = 1.0 / 720.0
    for coef in (1.0 / 120.0, 1.0 / 24.0, 1.0 / 6.0, 0.5, 1.0, 1.0):
        poly = poly * w + coef
    return poly * p2k


def _log_via_exp(z):
    zi = lax.bitcast_convert_type(z, jnp.int32)
    e = (lax.shift_right_logical(zi, 23) - 127).astype(jnp.float32)
    mant = lax.bitcast_convert_type((zi & 0x007FFFFF) | 0x3F800000, jnp.float32)
    a = (mant - 1.0) / (mant + 1.0)
    a2 = a * a
    y = e * _LN2 + 2.0 * a * (
        1.0 + a2 * (1.0 / 3.0 + a2 * (0.2 + a2 * (1.0 / 7.0)))
    )
    y = y + z * _fast_exp(-y) - 1.0
    return y


def _make_sc_call(num_subcores: int):
    rows_per_w = _B // num_subcores
    chunks = rows_per_w // _CHUNK
    strips = _CHUNK // 16

    mesh = plsc.VectorSubcoreMesh(
        core_axis_name="c", subcore_axis_name="s", num_cores=1
    )

    @functools.partial(
        pl.kernel,
        mesh=mesh,
        compiler_params=pltpu.CompilerParams(needs_layout_passes=False),
        out_type=jax.ShapeDtypeStruct((1,), jnp.float32),
        scratch_types=[
            pltpu.VMEM((2, _CHUNK, _C), jnp.float32),
            pltpu.VMEM((2, _CHUNK, _C), jnp.int32),
            pltpu.VMEM((1, _C), jnp.float32),
            pltpu.VMEM((16,), jnp.float32),
            pltpu.VMEM((num_subcores * 16,), jnp.float32),
            pltpu.VMEM_SHARED((num_subcores * 16,), jnp.float32),
            pltpu.SemaphoreType.DMA,
            pltpu.SemaphoreType.DMA,
        ],
    )
    def ldam_sc(
        x_hbm, t_hbm, m_hbm, out_hbm, xv, tv, mv, stage, gath, shared, sem0, sem1
    ):
        sid = lax.axis_index("s")
        row0 = sid * rows_per_w
        pltpu.sync_copy(m_hbm.at[pl.ds(15, 1), :], mv)

        lane = lax.iota(jnp.int32, 16)
        zero16 = jnp.zeros((16,), jnp.int32)
        diag = [(lane + j) & 15 for j in range(_C)]
        sems = [sem0, sem1]

        def start_chunk(k):
            b = k % 2
            base = row0 + k * _CHUNK
            pltpu.async_copy(
                x_hbm.at[pl.ds(base, _CHUNK), :], xv.at[b], sems[b]
            )
            pltpu.async_copy(
                t_hbm.at[pl.ds(base, _CHUNK), :], tv.at[b], sems[b]
            )

        def wait_chunk(k):
            b = k % 2
            pltpu.make_async_copy(
                x_hbm.at[pl.ds(row0, _CHUNK), :], xv.at[b], sems[b]
            ).wait()
            pltpu.make_async_copy(
                t_hbm.at[pl.ds(row0, _CHUNK), :], tv.at[b], sems[b]
            ).wait()

        def strip_body(i, acc, xvb, tvb):
            rows = i * 16 + lane
            c = plsc.load_gather(tvb, [rows, zero16 + 15]) * 2
            ms = plsc.load_gather(mv, [zero16, c]) * _S
            zero = jnp.zeros((16,), jnp.float32)
            dxr = zero
            ssum = zero
            xs = []
            for j in range(_C):
                xj = plsc.load_gather(xvb, [rows, diag[j]])
                tj = plsc.load_gather(tvb, [rows, diag[j]])
                xs.append(xj)
                tf = tj.astype(jnp.float32)
                dxr = dxr + tf * xj
                ssum = ssum + tf
            mx = xs[0]
            for j in range(1, _C):
                mx = jnp.maximum(mx, xs[j])
            mx = mx * _S
            z = zero
            for j in range(_C):
                z = z + _fast_exp(xs[j] * _S - mx)
            xc = plsc.load_gather(xvb, [rows, c])
            tc = plsc.load_gather(tvb, [rows, c]).astype(jnp.float32)
            z = z + _fast_exp(xc * _S - mx) * (_fast_exp(-ms) - 1.0)
            lse = mx + _log_via_exp(z)
            return acc + (ssum * lse - _S * dxr + tc * ms)

        start_chunk(0)
        acc = jnp.zeros((16,), jnp.float32)
        for k in range(chunks):
            wait_chunk(k)
            if k + 1 < chunks:
                start_chunk(k + 1)
            b = k % 2
            xvb, tvb = xv.at[b], tv.at[b]

            @plsc.parallel_loop(0, strips, carry=acc)
            def acc_loop(i, a, xvb=xvb, tvb=tvb):
                return strip_body(i, a, xvb, tvb)

            acc = acc_loop
        stage[...] = acc * (2.0 / _B)
        pltpu.sync_copy(stage, shared.at[pl.ds(sid * 16, 16)])
        plsc.subcore_barrier()

        @pl.when(sid == 0)
        def _():
            pltpu.sync_copy(shared, gath)
            tot = gath[pl.ds(0, 16)]
            for w in range(1, num_subcores):
                tot = tot + gath[pl.ds(w * 16, 16)]
            total = jnp.sum(tot)
            stage[...] = jnp.broadcast_to(total, (16,))
            pltpu.sync_copy(stage.at[pl.ds(0, 1)], out_hbm)

    return ldam_sc


def kernel(x, target, m_list):
    info = plsc.get_sparse_core_info()
    sc_call = _make_sc_call(info.num_subcores)
    out = sc_call(x, target, m_list)
    return out.reshape(())


# speedup vs baseline: 1.0285x; 1.0285x over previous
import functools

import jax
import jax.numpy as jnp
from jax import lax
from jax.experimental import pallas as pl
from jax.experimental.pallas import tpu as pltpu
from jax.experimental.pallas import tpu_sc as plsc

_B = 16384
_C = 16
_S = 30.0
_LN2 = 0.6931471805599453
_CHUNK = 128


def _log_via_exp(z):
    zi = lax.bitcast_convert_type(z, jnp.int32)
    e = (lax.shift_right_logical(zi, 23) - 127).astype(jnp.float32)
    mant = lax.bitcast_convert_type((zi & 0x007FFFFF) | 0x3F800000, jnp.float32)
    a = (mant - 1.0) / (mant + 1.0)
    a2 = a * a
    y = e * _LN2 + 2.0 * a * (
        1.0 + a2 * (1.0 / 3.0 + a2 * (0.2 + a2 * (1.0 / 7.0)))
    )
    y = y + z * jnp.exp(-y) - 1.0
    return y


def _make_sc_call(num_subcores: int):
    rows_per_w = _B // num_subcores
    chunks = rows_per_w // _CHUNK
    strips = _CHUNK // 16

    mesh = plsc.VectorSubcoreMesh(
        core_axis_name="c", subcore_axis_name="s", num_cores=1
    )

    @functools.partial(
        pl.kernel,
        mesh=mesh,
        compiler_params=pltpu.CompilerParams(needs_layout_passes=False),
        out_type=jax.ShapeDtypeStruct((1,), jnp.float32),
        scratch_types=[
            pltpu.VMEM((2, _CHUNK, _C), jnp.float32),
            pltpu.VMEM((2, _CHUNK, _C), jnp.int32),
            pltpu.VMEM((1, _C), jnp.float32),
            pltpu.VMEM((16,), jnp.float32),
            pltpu.VMEM((num_subcores * 16,), jnp.float32),
            pltpu.VMEM_SHARED((num_subcores * 16,), jnp.float32),
            pltpu.SemaphoreType.DMA,
            pltpu.SemaphoreType.DMA,
        ],
    )
    def ldam_sc(
        x_hbm, t_hbm, m_hbm, out_hbm, xv, tv, mv, stage, gath, shared, sem0, sem1
    ):
        sid = lax.axis_index("s")
        row0 = sid * rows_per_w
        pltpu.sync_copy(m_hbm.at[pl.ds(15, 1), :], mv)

        lane = lax.iota(jnp.int32, 16)
        zero16 = jnp.zeros((16,), jnp.int32)
        diag = [(lane + j) & 15 for j in range(_C)]
        sems = [sem0, sem1]

        def start_chunk(k):
            b = k % 2
            base = row0 + k * _CHUNK
            pltpu.async_copy(
                x_hbm.at[pl.ds(base, _CHUNK), :], xv.at[b], sems[b]
            )
            pltpu.async_copy(
                t_hbm.at[pl.ds(base, _CHUNK), :], tv.at[b], sems[b]
            )

        def wait_chunk(k):
            b = k % 2
            pltpu.make_async_copy(
                x_hbm.at[pl.ds(row0, _CHUNK), :], xv.at[b], sems[b]
            ).wait()
            pltpu.make_async_copy(
                t_hbm.at[pl.ds(row0, _CHUNK), :], tv.at[b], sems[b]
            ).wait()

        def strip_body(i, acc, xvb, tvb):
            rows = i * 16 + lane
            c = plsc.load_gather(tvb, [rows, zero16 + 15]) * 2
            ms = plsc.load_gather(mv, [zero16, c]) * _S
            zero = jnp.zeros((16,), jnp.float32)
            dxr = zero
            ssum = zero
            xs = []
            for j in range(_C):
                xj = plsc.load_gather(xvb, [rows, diag[j]])
                tj = plsc.load_gather(tvb, [rows, diag[j]])
                xs.append(xj)
                tf = tj.astype(jnp.float32)
                dxr = dxr + tf * xj
                ssum = ssum + tf
            mx = xs[0]
            for j in range(1, _C):
                mx = jnp.maximum(mx, xs[j])
            mx = mx * _S
            z = zero
            for j in range(_C):
                z = z + jnp.exp(xs[j] * _S - mx)
            xc = plsc.load_gather(xvb, [rows, c])
            tc = plsc.load_gather(tvb, [rows, c]).astype(jnp.float32)
            z = z + jnp.exp(xc * _S - mx) * (jnp.exp(-ms) - 1.0)
            lse = mx + _log_via_exp(z)
            return acc + (ssum * lse - _S * dxr + tc * ms)

        start_chunk(0)
        acc = jnp.zeros((16,), jnp.float32)
        for k in range(chunks):
            wait_chunk(k)
            if k + 1 < chunks:
                start_chunk(k + 1)
            b = k % 2
            xvb, tvb = xv.at[b], tv.at[b]

            @plsc.parallel_loop(0, strips, carry=acc)
            def acc_loop(i, a, xvb=xvb, tvb=tvb):
                return strip_body(i, a, xvb, tvb)

            acc = acc_loop
        stage[...] = acc * (2.0 / _B)
        pltpu.sync_copy(stage, shared.at[pl.ds(sid * 16, 16)])
        plsc.subcore_barrier()

        @pl.when(sid == 0)
        def _():
            pltpu.sync_copy(shared, gath)
            tot = gath[pl.ds(0, 16)]
            for w in range(1, num_subcores):
                tot = tot + gath[pl.ds(w * 16, 16)]
            total = jnp.sum(tot)
            stage[...] = jnp.broadcast_to(total, (16,))
            pltpu.sync_copy(stage.at[pl.ds(0, 1)], out_hbm)

    return ldam_sc


def kernel(x, target, m_list):
    info = plsc.get_sparse_core_info()
    sc_call = _make_sc_call(info.num_subcores)
    out = sc_call(x, target, m_list)
    return out.reshape(())
